# Initial kernel scaffold; baseline (speedup 1.0000x reference)
import jax
import jax.numpy as jnp
from jax.experimental import pallas as pl


def kernel(xyz, features, temb, W, gamma, beta):
    raise NotImplementedError("write your pallas kernel here")


# speedup vs baseline: 12.7576x; 12.7576x over previous
import functools

import jax
import jax.numpy as jnp
from jax.experimental import pallas as pl

_STRIDE = 4
_K = 16
_CPAD = 128
_TS = 128
_EPS = 1e-5


def _fps_body(xs_ref, ys_ref, zs_ref, nx_ref, ny_ref, nz_ref, *, n, s):
    b, rows, _ = xs_ref.shape
    srows = nx_ref.shape[1]
    flat = (jax.lax.broadcasted_iota(jnp.int32, (rows, 128), 0) * 128
            + jax.lax.broadcasted_iota(jnp.int32, (rows, 128), 1))[None]
    pos = (jax.lax.broadcasted_iota(jnp.int32, (srows, 128), 0) * 128
           + jax.lax.broadcasted_iota(jnp.int32, (srows, 128), 1))[None]
    xs = xs_ref[...]
    ys = ys_ref[...]
    zs = zs_ref[...]

    def body(i, st):
        dists, far, bx, by, bz = st
        fm = flat == far
        cx = jnp.sum(jnp.where(fm, xs, 0.0), axis=(1, 2), keepdims=True)
        cy = jnp.sum(jnp.where(fm, ys, 0.0), axis=(1, 2), keepdims=True)
        cz = jnp.sum(jnp.where(fm, zs, 0.0), axis=(1, 2), keepdims=True)
        sm = pos == i
        bx = jnp.where(sm, cx, bx)
        by = jnp.where(sm, cy, by)
        bz = jnp.where(sm, cz, bz)
        d = (xs - cx) ** 2 + (ys - cy) ** 2 + (zs - cz) ** 2
        dists = jnp.minimum(dists, d)
        m = jnp.max(dists, axis=(1, 2), keepdims=True)
        far = jnp.min(jnp.where(dists == m, flat, n), axis=(1, 2), keepdims=True)
        return dists, far, bx, by, bz

    dists0 = jnp.full((b, rows, 128), 1e10, jnp.float32)
    far0 = jnp.zeros((b, 1, 1), jnp.int32)
    z = jnp.zeros((b, srows, 128), jnp.float32)
    _, _, bx, by, bz = jax.lax.fori_loop(0, s, body, (dists0, far0, z, z, z))
    nx_ref[...] = bx
    ny_ref[...] = by
    nz_ref[...] = bz


def _fps(xs, ys, zs, n, s):
    b = xs.shape[0]
    srows = s // 128
    out_sd = jax.ShapeDtypeStruct((b, srows, 128), jnp.float32)
    return pl.pallas_call(
        functools.partial(_fps_body, n=n, s=s),
        out_shape=[out_sd, out_sd, out_sd],
    )(xs, ys, zs)


def _lin_body(xf_ref, wt_ref, g_ref):
    g_ref[0] = jnp.dot(xf_ref[0], wt_ref[...],
                       preferred_element_type=jnp.float32)


def _lin(xf_pad, wt_pad):
    b, n, _ = xf_pad.shape
    return pl.pallas_call(
        _lin_body,
        grid=(b,),
        in_specs=[
            pl.BlockSpec((1, n, _CPAD), lambda i: (i, 0, 0)),
            pl.BlockSpec((_CPAD, _CPAD), lambda i: (0, 0)),
        ],
        out_specs=pl.BlockSpec((1, n, _CPAD), lambda i: (i, 0, 0)),
        out_shape=jax.ShapeDtypeStruct((b, n, _CPAD), jnp.float32),
    )(xf_pad, wt_pad)


def _knn_body(nx_ref, ny_ref, nz_ref, xs_ref, ys_ref, zs_ref, g_ref, wt_ref,
              maxl_ref, minl_ref, stats_ref, *, n, k, ts):
    b = pl.program_id(0)
    st = pl.program_id(1)
    qx = nx_ref[0]
    qy = ny_ref[0]
    qz = nz_ref[0]
    xs = xs_ref[0]
    ys = ys_ref[0]
    zs = zs_ref[0]
    g = g_ref[0]
    qw = (qx * wt_ref[0:1, :] + qy * wt_ref[1:2, :]
          + qz * wt_ref[2:3, :])
    ii = jax.lax.broadcasted_iota(jnp.int32, (ts, n), 1)
    d0 = (qx - xs) ** 2 + (qy - ys) ** 2 + (qz - zs) ** 2

    def body(j, st_c):
        d, mx, mn, sm, sq = st_c
        m = jnp.min(d, axis=1, keepdims=True)
        idxv = jnp.min(jnp.where(d == m, ii, n), axis=1, keepdims=True)
        oh = ii == idxv
        lin = jnp.dot(oh.astype(jnp.float32), g,
                      preferred_element_type=jnp.float32) - qw
        mx = jnp.maximum(mx, lin)
        mn = jnp.minimum(mn, lin)
        sm = sm + lin
        sq = sq + lin * lin
        d = jnp.where(oh, jnp.inf, d)
        return d, mx, mn, sm, sq

    mx0 = jnp.full((ts, _CPAD), -jnp.inf, jnp.float32)
    mn0 = jnp.full((ts, _CPAD), jnp.inf, jnp.float32)
    z = jnp.zeros((ts, _CPAD), jnp.float32)
    _, mx, mn, sm, sq = jax.lax.fori_loop(0, k, body, (d0, mx0, mn0, z, z))

    maxl_ref[0] = mx
    minl_ref[0] = mn

    @pl.when(jnp.logical_and(b == 0, st == 0))
    def _():
        stats_ref[...] = jnp.zeros((8, _CPAD), jnp.float32)

    stats_ref[0:1, :] += jnp.sum(sm, axis=0, keepdims=True)
    stats_ref[1:2, :] += jnp.sum(sq, axis=0, keepdims=True)


def _knn(nx3, ny3, nz3, xs3, ys3, zs3, g, wt_pad, n, s, k):
    b = g.shape[0]
    ts = _TS
    grid = (b, s // ts)
    q_spec = pl.BlockSpec((1, ts, 1), lambda i, j: (i, j, 0))
    p_spec = pl.BlockSpec((1, 1, n), lambda i, j: (i, 0, 0))
    g_spec = pl.BlockSpec((1, n, _CPAD), lambda i, j: (i, 0, 0))
    w_spec = pl.BlockSpec((_CPAD, _CPAD), lambda i, j: (0, 0))
    o_spec = pl.BlockSpec((1, ts, _CPAD), lambda i, j: (i, j, 0))
    s_spec = pl.BlockSpec((8, _CPAD), lambda i, j: (0, 0))
    return pl.pallas_call(
        functools.partial(_knn_body, n=n, k=k, ts=ts),
        grid=grid,
        in_specs=[q_spec, q_spec, q_spec, p_spec, p_spec, p_spec,
                  g_spec, w_spec],
        out_specs=[o_spec, o_spec, s_spec],
        out_shape=[
            jax.ShapeDtypeStruct((b, s, _CPAD), jnp.float32),
            jax.ShapeDtypeStruct((b, s, _CPAD), jnp.float32),
            jax.ShapeDtypeStruct((8, _CPAD), jnp.float32),
        ],
    )(nx3, ny3, nz3, xs3, ys3, zs3, g, wt_pad)


def _fin_body(maxl_ref, minl_ref, stats_ref, gb_ref, out_ref, *, cnt):
    mean = stats_ref[0:1, :] / cnt
    ex2 = stats_ref[1:2, :] / cnt
    var = ex2 - mean * mean
    gamma = gb_ref[0:1, :]
    beta = gb_ref[1:2, :]
    inv = jax.lax.rsqrt(var + _EPS)
    scale = gamma * inv
    shift = beta - mean * scale
    ext = jnp.where(gamma >= 0.0, maxl_ref[0], minl_ref[0])
    out_ref[0] = jnp.maximum(ext * scale + shift, 0.0)


def _finalize(maxl, minl, stats, gb, cnt):
    b, s, _ = maxl.shape
    m_spec = pl.BlockSpec((1, s, _CPAD), lambda i: (i, 0, 0))
    s_spec = pl.BlockSpec((8, _CPAD), lambda i: (0, 0))
    return pl.pallas_call(
        functools.partial(_fin_body, cnt=cnt),
        grid=(b,),
        in_specs=[m_spec, m_spec, s_spec, s_spec],
        out_specs=m_spec,
        out_shape=jax.ShapeDtypeStruct((b, s, _CPAD), jnp.float32),
    )(maxl, minl, stats, gb)


def kernel(xyz, features, temb, W, gamma, beta):
    b, n, _ = xyz.shape
    cin = features.shape[-1]
    cout = W.shape[0]
    s = n // _STRIDE
    rows = n // 128

    xs = xyz[:, :, 0].reshape(b, rows, 128)
    ys = xyz[:, :, 1].reshape(b, rows, 128)
    zs = xyz[:, :, 2].reshape(b, rows, 128)
    nx, ny, nz = _fps(xs, ys, zs, n, s)
    new_xyz = jnp.stack(
        [nx.reshape(b, s), ny.reshape(b, s), nz.reshape(b, s)], axis=-1)

    pad = jnp.zeros((b, n, _CPAD - 3 - cin), jnp.float32)
    xf = jnp.concatenate([xyz, features, pad], axis=-1)
    wt_pad = jnp.concatenate(
        [W.T, jnp.zeros((_CPAD - 3 - cin, cout), jnp.float32)], axis=0)
    wt_pad = jnp.concatenate(
        [wt_pad, jnp.zeros((_CPAD, _CPAD - cout), jnp.float32)], axis=1)
    g = _lin(xf, wt_pad)

    nx3 = nx.reshape(b, s, 1)
    ny3 = ny.reshape(b, s, 1)
    nz3 = nz.reshape(b, s, 1)
    xs3 = xyz[:, :, 0].reshape(b, 1, n)
    ys3 = xyz[:, :, 1].reshape(b, 1, n)
    zs3 = xyz[:, :, 2].reshape(b, 1, n)
    maxl, minl, stats = _knn(nx3, ny3, nz3, xs3, ys3, zs3, g, wt_pad, n, s, _K)

    gb = jnp.concatenate(
        [gamma.reshape(1, -1), beta.reshape(1, -1),
         jnp.zeros((6, cout), jnp.float32)], axis=0)
    gb = jnp.concatenate(
        [gb, jnp.zeros((8, _CPAD - cout), jnp.float32)], axis=1)
    cnt = float(b * s * _K)
    feats_out = _finalize(maxl, minl, stats, gb, cnt)[:, :, :cout]

    return new_xyz, feats_out, temb[:, :s, :]


# speedup vs baseline: 13.9441x; 1.0930x over previous
import functools

import jax
import jax.numpy as jnp
from jax.experimental import pallas as pl
from jax.experimental.pallas import tpu as pltpu
from jax.experimental.pallas import tpu_sc as plsc

_STRIDE = 4
_K = 16
_CPAD = 128
_TS = 128
_EPS = 1e-5


def _fps_body(xs_ref, ys_ref, zs_ref, nx_ref, ny_ref, nz_ref, *, n, s):
    b, rows, _ = xs_ref.shape
    srows = nx_ref.shape[1]
    flat = (jax.lax.broadcasted_iota(jnp.int32, (rows, 128), 0) * 128
            + jax.lax.broadcasted_iota(jnp.int32, (rows, 128), 1))[None]
    pos = (jax.lax.broadcasted_iota(jnp.int32, (srows, 128), 0) * 128
           + jax.lax.broadcasted_iota(jnp.int32, (srows, 128), 1))[None]
    xs = xs_ref[...]
    ys = ys_ref[...]
    zs = zs_ref[...]

    def body(i, st):
        dists, far, bx, by, bz = st
        fm = flat == far
        cx = jnp.sum(jnp.where(fm, xs, 0.0), axis=(1, 2), keepdims=True)
        cy = jnp.sum(jnp.where(fm, ys, 0.0), axis=(1, 2), keepdims=True)
        cz = jnp.sum(jnp.where(fm, zs, 0.0), axis=(1, 2), keepdims=True)
        sm = pos == i
        bx = jnp.where(sm, cx, bx)
        by = jnp.where(sm, cy, by)
        bz = jnp.where(sm, cz, bz)
        d = (xs - cx) ** 2 + (ys - cy) ** 2 + (zs - cz) ** 2
        dists = jnp.minimum(dists, d)
        m = jnp.max(dists, axis=(1, 2), keepdims=True)
        far = jnp.min(jnp.where(dists == m, flat, n), axis=(1, 2), keepdims=True)
        return dists, far, bx, by, bz

    dists0 = jnp.full((b, rows, 128), 1e10, jnp.float32)
    far0 = jnp.zeros((b, 1, 1), jnp.int32)
    z = jnp.zeros((b, srows, 128), jnp.float32)
    _, _, bx, by, bz = jax.lax.fori_loop(0, s, body, (dists0, far0, z, z, z))
    nx_ref[...] = bx
    ny_ref[...] = by
    nz_ref[...] = bz


def _fps(xs, ys, zs, n, s):
    b = xs.shape[0]
    srows = s // 128
    out_sd = jax.ShapeDtypeStruct((b, srows, 128), jnp.float32)
    return pl.pallas_call(
        functools.partial(_fps_body, n=n, s=s),
        out_shape=[out_sd, out_sd, out_sd],
    )(xs, ys, zs)


def _lin_body(xf_ref, wt_ref, g_ref):
    g_ref[0] = jnp.dot(xf_ref[0], wt_ref[...],
                       preferred_element_type=jnp.float32)


def _lin(xf_pad, wt_pad):
    b, n, _ = xf_pad.shape
    return pl.pallas_call(
        _lin_body,
        grid=(b,),
        in_specs=[
            pl.BlockSpec((1, n, _CPAD), lambda i: (i, 0, 0)),
            pl.BlockSpec((_CPAD, _CPAD), lambda i: (0, 0)),
        ],
        out_specs=pl.BlockSpec((1, n, _CPAD), lambda i: (i, 0, 0)),
        out_shape=jax.ShapeDtypeStruct((b, n, _CPAD), jnp.float32),
    )(xf_pad, wt_pad)


def _knn_body(nx_ref, ny_ref, nz_ref, xs_ref, ys_ref, zs_ref, idx_ref,
              *, n, k, ts):
    b = pl.program_id(0)
    qx = nx_ref[0]
    qy = ny_ref[0]
    qz = nz_ref[0]
    xs = xs_ref[0]
    ys = ys_ref[0]
    zs = zs_ref[0]
    ii = jax.lax.broadcasted_iota(jnp.int32, (ts, n), 1)
    kk = jax.lax.broadcasted_iota(jnp.int32, (ts, k), 1)
    d0 = (qx - xs) ** 2 + (qy - ys) ** 2 + (qz - zs) ** 2

    def body(j, st_c):
        d, acc = st_c
        m = jnp.min(d, axis=1, keepdims=True)
        idxv = jnp.min(jnp.where(d == m, ii, n), axis=1, keepdims=True)
        acc = jnp.where(kk == j, idxv, acc)
        d = jnp.where(ii == idxv, jnp.inf, d)
        return d, acc

    acc0 = jnp.zeros((ts, k), jnp.int32)
    _, acc = jax.lax.fori_loop(0, k, body, (d0, acc0))
    idx_ref[0] = acc + b * n


def _knn(nx3, ny3, nz3, xs3, ys3, zs3, n, s, k):
    b = nx3.shape[0]
    ts = _TS
    grid = (b, s // ts)
    q_spec = pl.BlockSpec((1, ts, 1), lambda i, j: (i, j, 0))
    p_spec = pl.BlockSpec((1, 1, n), lambda i, j: (i, 0, 0))
    o_spec = pl.BlockSpec((1, ts, k), lambda i, j: (i, j, 0))
    return pl.pallas_call(
        functools.partial(_knn_body, n=n, k=k, ts=ts),
        grid=grid,
        in_specs=[q_spec, q_spec, q_spec, p_spec, p_spec, p_spec],
        out_specs=o_spec,
        out_shape=jax.ShapeDtypeStruct((b, s, k), jnp.int32),
    )(nx3, ny3, nz3, xs3, ys3, zs3)


def _sc_gather_build(bs, k):
    info = plsc.get_sparse_core_info()
    nc, ns = info.num_cores, info.num_subcores
    nw = nc * ns
    qpw = bs // nw
    chunk = 8
    nchunks = qpw // chunk
    mesh = plsc.VectorSubcoreMesh(core_axis_name="c", subcore_axis_name="s")
    out_sd = jax.ShapeDtypeStruct((bs, _CPAD), jnp.float32)

    @functools.partial(
        pl.kernel, mesh=mesh,
        out_type=[out_sd, out_sd, out_sd, out_sd],
        scratch_types=[
            pltpu.VMEM((qpw * k,), jnp.int32),
            pltpu.VMEM((chunk * k, _CPAD), jnp.float32),
            pltpu.VMEM((chunk, _CPAD), jnp.float32),
            pltpu.VMEM((chunk, _CPAD), jnp.float32),
            pltpu.VMEM((chunk, _CPAD), jnp.float32),
            pltpu.VMEM((chunk, _CPAD), jnp.float32),
            pltpu.SemaphoreType.DMA,
        ],
    )
    def sck(g_hbm, idx_hbm, mx_hbm, mn_hbm, sm_hbm, sq_hbm,
            idx_v, rows_v, omx, omn, osm, osq, sem):
        wid = jax.lax.axis_index("s") * nc + jax.lax.axis_index("c")
        baseq = wid * qpw
        pltpu.sync_copy(idx_hbm.at[pl.ds(baseq * k, qpw * k)], idx_v)

        def chunk_body(ch, carry):
            pltpu.async_copy(
                g_hbm.at[idx_v.at[pl.ds(ch * (chunk * k), chunk * k)]],
                rows_v, sem).wait()

            def q_body(q, c2):
                rbase = q * k
                for c in range(_CPAD // 16):
                    sl = pl.ds(c * 16, 16)
                    v = rows_v[rbase, sl]
                    mx = v
                    mn = v
                    sm = v
                    sq = v * v
                    for r in range(1, k):
                        v = rows_v[rbase + r, sl]
                        mx = jnp.maximum(mx, v)
                        mn = jnp.minimum(mn, v)
                        sm = sm + v
                        sq = sq + v * v
                    omx[q, sl] = mx
                    omn[q, sl] = mn
                    osm[q, sl] = sm
                    osq[q, sl] = sq
                return c2

            jax.lax.fori_loop(0, chunk, q_body, 0)
            row0 = baseq + ch * chunk
            pltpu.sync_copy(omx, mx_hbm.at[pl.ds(row0, chunk)])
            pltpu.sync_copy(omn, mn_hbm.at[pl.ds(row0, chunk)])
            pltpu.sync_copy(osm, sm_hbm.at[pl.ds(row0, chunk)])
            pltpu.sync_copy(osq, sq_hbm.at[pl.ds(row0, chunk)])
            return carry

        jax.lax.fori_loop(0, nchunks, chunk_body, 0)

    return sck


def _stats_body(sm_ref, sq_ref, nx_ref, ny_ref, nz_ref, wt_ref, st_ref, *, k):
    b = pl.program_id(0)
    qw = (nx_ref[0] * wt_ref[0:1, :] + ny_ref[0] * wt_ref[1:2, :]
          + nz_ref[0] * wt_ref[2:3, :])
    sm = sm_ref[0]
    sq = sq_ref[0]
    lin_sum = sm - k * qw
    lin_sq = sq - 2.0 * qw * sm + k * (qw * qw)

    @pl.when(b == 0)
    def _():
        st_ref[...] = jnp.zeros((8, _CPAD), jnp.float32)

    st_ref[0:1, :] += jnp.sum(lin_sum, axis=0, keepdims=True)
    st_ref[1:2, :] += jnp.sum(lin_sq, axis=0, keepdims=True)


def _stats(smg, sqg, nx3, ny3, nz3, wt_pad, k):
    b, s, _ = smg.shape
    m_spec = pl.BlockSpec((1, s, _CPAD), lambda i: (i, 0, 0))
    q_spec = pl.BlockSpec((1, s, 1), lambda i: (i, 0, 0))
    w_spec = pl.BlockSpec((_CPAD, _CPAD), lambda i: (0, 0))
    s_spec = pl.BlockSpec((8, _CPAD), lambda i: (0, 0))
    return pl.pallas_call(
        functools.partial(_stats_body, k=float(k)),
        grid=(b,),
        in_specs=[m_spec, m_spec, q_spec, q_spec, q_spec, w_spec],
        out_specs=s_spec,
        out_shape=jax.ShapeDtypeStruct((8, _CPAD), jnp.float32),
    )(smg, sqg, nx3, ny3, nz3, wt_pad)


def _apply_body(mx_ref, mn_ref, nx_ref, ny_ref, nz_ref, wt_ref, st_ref,
                gb_ref, out_ref, *, cnt):
    qw = (nx_ref[0] * wt_ref[0:1, :] + ny_ref[0] * wt_ref[1:2, :]
          + nz_ref[0] * wt_ref[2:3, :])
    mean = st_ref[0:1, :] / cnt
    ex2 = st_ref[1:2, :] / cnt
    var = ex2 - mean * mean
    gamma = gb_ref[0:1, :]
    beta = gb_ref[1:2, :]
    inv = jax.lax.rsqrt(var + _EPS)
    scale = gamma * inv
    shift = beta - mean * scale
    ext = jnp.where(gamma >= 0.0, mx_ref[0], mn_ref[0]) - qw
    out_ref[0] = jnp.maximum(ext * scale + shift, 0.0)


def _apply(mxg, mng, nx3, ny3, nz3, wt_pad, stats, gb, cnt):
    b, s, _ = mxg.shape
    m_spec = pl.BlockSpec((1, s, _CPAD), lambda i: (i, 0, 0))
    q_spec = pl.BlockSpec((1, s, 1), lambda i: (i, 0, 0))
    w_spec = pl.BlockSpec((_CPAD, _CPAD), lambda i: (0, 0))
    s_spec = pl.BlockSpec((8, _CPAD), lambda i: (0, 0))
    return pl.pallas_call(
        functools.partial(_apply_body, cnt=cnt),
        grid=(b,),
        in_specs=[m_spec, m_spec, q_spec, q_spec, q_spec, w_spec,
                  s_spec, s_spec],
        out_specs=m_spec,
        out_shape=jax.ShapeDtypeStruct((b, s, _CPAD), jnp.float32),
    )(mxg, mng, nx3, ny3, nz3, wt_pad, stats, gb)


def kernel(xyz, features, temb, W, gamma, beta):
    b, n, _ = xyz.shape
    cin = features.shape[-1]
    cout = W.shape[0]
    s = n // _STRIDE
    rows = n // 128

    xs = xyz[:, :, 0].reshape(b, rows, 128)
    ys = xyz[:, :, 1].reshape(b, rows, 128)
    zs = xyz[:, :, 2].reshape(b, rows, 128)
    nx, ny, nz = _fps(xs, ys, zs, n, s)
    new_xyz = jnp.stack(
        [nx.reshape(b, s), ny.reshape(b, s), nz.reshape(b, s)], axis=-1)

    pad = jnp.zeros((b, n, _CPAD - 3 - cin), jnp.float32)
    xf = jnp.concatenate([xyz, features, pad], axis=-1)
    wt_pad = jnp.concatenate(
        [W.T, jnp.zeros((_CPAD - 3 - cin, cout), jnp.float32)], axis=0)
    wt_pad = jnp.concatenate(
        [wt_pad, jnp.zeros((_CPAD, _CPAD - cout), jnp.float32)], axis=1)
    g = _lin(xf, wt_pad)

    nx3 = nx.reshape(b, s, 1)
    ny3 = ny.reshape(b, s, 1)
    nz3 = nz.reshape(b, s, 1)
    xs3 = xyz[:, :, 0].reshape(b, 1, n)
    ys3 = xyz[:, :, 1].reshape(b, 1, n)
    zs3 = xyz[:, :, 2].reshape(b, 1, n)
    knn_idx = _knn(nx3, ny3, nz3, xs3, ys3, zs3, n, s, _K)

    sck = _sc_gather_build(b * s, _K)
    mxg, mng, smg, sqg = sck(g.reshape(b * n, _CPAD),
                             knn_idx.reshape(b * s * _K))
    mxg = mxg.reshape(b, s, _CPAD)
    mng = mng.reshape(b, s, _CPAD)
    smg = smg.reshape(b, s, _CPAD)
    sqg = sqg.reshape(b, s, _CPAD)

    stats = _stats(smg, sqg, nx3, ny3, nz3, wt_pad, _K)
    gb = jnp.concatenate(
        [gamma.reshape(1, -1), beta.reshape(1, -1),
         jnp.zeros((6, cout), jnp.float32)], axis=0)
    gb = jnp.concatenate(
        [gb, jnp.zeros((8, _CPAD - cout), jnp.float32)], axis=1)
    cnt = float(b * s * _K)
    feats_out = _apply(mxg, mng, nx3, ny3, nz3, wt_pad, stats, gb,
                       cnt)[:, :, :cout]

    return new_xyz, feats_out, temb[:, :s, :]


# speedup vs baseline: 14.8464x; 1.0647x over previous
import functools

import jax
import jax.numpy as jnp
from jax.experimental import pallas as pl
from jax.experimental.pallas import tpu as pltpu
from jax.experimental.pallas import tpu_sc as plsc

_STRIDE = 4
_K = 16
_CPAD = 128
_TS = 128
_EPS = 1e-5


def _fps_body(xs_ref, ys_ref, zs_ref, nx_ref, ny_ref, nz_ref, *, n, s):
    b, rows, _ = xs_ref.shape
    srows = nx_ref.shape[1]
    flat = (jax.lax.broadcasted_iota(jnp.int32, (rows, 128), 0) * 128
            + jax.lax.broadcasted_iota(jnp.int32, (rows, 128), 1))[None]
    pos = (jax.lax.broadcasted_iota(jnp.int32, (srows, 128), 0) * 128
           + jax.lax.broadcasted_iota(jnp.int32, (srows, 128), 1))[None]
    xs = xs_ref[...]
    ys = ys_ref[...]
    zs = zs_ref[...]

    def body(i, st):
        dists, far, bx, by, bz = st
        fm = flat == far
        cx = jnp.sum(jnp.where(fm, xs, 0.0), axis=(1, 2), keepdims=True)
        cy = jnp.sum(jnp.where(fm, ys, 0.0), axis=(1, 2), keepdims=True)
        cz = jnp.sum(jnp.where(fm, zs, 0.0), axis=(1, 2), keepdims=True)
        sm = pos == i
        bx = jnp.where(sm, cx, bx)
        by = jnp.where(sm, cy, by)
        bz = jnp.where(sm, cz, bz)
        d = (xs - cx) ** 2 + (ys - cy) ** 2 + (zs - cz) ** 2
        dists = jnp.minimum(dists, d)
        m = jnp.max(dists, axis=(1, 2), keepdims=True)
        far = jnp.min(jnp.where(dists == m, flat, n), axis=(1, 2), keepdims=True)
        return dists, far, bx, by, bz

    dists0 = jnp.full((b, rows, 128), 1e10, jnp.float32)
    far0 = jnp.zeros((b, 1, 1), jnp.int32)
    z = jnp.zeros((b, srows, 128), jnp.float32)
    _, _, bx, by, bz = jax.lax.fori_loop(0, s, body, (dists0, far0, z, z, z))
    nx_ref[...] = bx
    ny_ref[...] = by
    nz_ref[...] = bz


def _fps(xs, ys, zs, n, s):
    b = xs.shape[0]
    srows = s // 128
    out_sd = jax.ShapeDtypeStruct((b, srows, 128), jnp.float32)
    return pl.pallas_call(
        functools.partial(_fps_body, n=n, s=s),
        out_shape=[out_sd, out_sd, out_sd],
    )(xs, ys, zs)


def _lin_body(xf_ref, wt_ref, g_ref):
    g_ref[0] = jnp.dot(xf_ref[0], wt_ref[...],
                       preferred_element_type=jnp.float32)


def _lin(xf_pad, wt_pad):
    b, n, _ = xf_pad.shape
    return pl.pallas_call(
        _lin_body,
        grid=(b,),
        in_specs=[
            pl.BlockSpec((1, n, _CPAD), lambda i: (i, 0, 0)),
            pl.BlockSpec((_CPAD, _CPAD), lambda i: (0, 0)),
        ],
        out_specs=pl.BlockSpec((1, n, _CPAD), lambda i: (i, 0, 0)),
        out_shape=jax.ShapeDtypeStruct((b, n, _CPAD), jnp.float32),
    )(xf_pad, wt_pad)


def _knn_body(nx_ref, ny_ref, nz_ref, xs_ref, ys_ref, zs_ref, idx_ref,
              *, n, k, ts):
    b = pl.program_id(0)
    qx = nx_ref[0]
    qy = ny_ref[0]
    qz = nz_ref[0]
    xs = xs_ref[0]
    ys = ys_ref[0]
    zs = zs_ref[0]
    ii = jax.lax.broadcasted_iota(jnp.int32, (ts, n), 1)
    kk = jax.lax.broadcasted_iota(jnp.int32, (ts, k), 1)
    d0 = (qx - xs) ** 2 + (qy - ys) ** 2 + (qz - zs) ** 2

    def body(j, st_c):
        d, prev, acc = st_c
        dm = jnp.where(ii == prev, jnp.inf, d)
        idxv = jnp.argmin(dm, axis=1, keepdims=True).astype(jnp.int32)
        acc = jnp.where(kk == j, idxv, acc)
        return dm, idxv, acc

    acc0 = jnp.zeros((ts, k), jnp.int32)
    prev0 = jnp.full((ts, 1), -1, jnp.int32)
    _, _, acc = jax.lax.fori_loop(0, k, body, (d0, prev0, acc0))
    idx_ref[0] = acc + b * n


def _knn(nx3, ny3, nz3, xs3, ys3, zs3, n, s, k):
    b = nx3.shape[0]
    ts = _TS
    grid = (b, s // ts)
    q_spec = pl.BlockSpec((1, ts, 1), lambda i, j: (i, j, 0))
    p_spec = pl.BlockSpec((1, 1, n), lambda i, j: (i, 0, 0))
    o_spec = pl.BlockSpec((1, ts, k), lambda i, j: (i, j, 0))
    return pl.pallas_call(
        functools.partial(_knn_body, n=n, k=k, ts=ts),
        grid=grid,
        in_specs=[q_spec, q_spec, q_spec, p_spec, p_spec, p_spec],
        out_specs=o_spec,
        out_shape=jax.ShapeDtypeStruct((b, s, k), jnp.int32),
    )(nx3, ny3, nz3, xs3, ys3, zs3)


def _sc_gather_build(bs, k):
    info = plsc.get_sparse_core_info()
    nc, ns = info.num_cores, info.num_subcores
    nw = nc * ns
    qpw = bs // nw
    chunk = 8
    nchunks = qpw // chunk
    mesh = plsc.VectorSubcoreMesh(core_axis_name="c", subcore_axis_name="s")
    out_sd = jax.ShapeDtypeStruct((bs, _CPAD), jnp.float32)

    @functools.partial(
        pl.kernel, mesh=mesh,
        out_type=[out_sd, out_sd, out_sd, out_sd],
        scratch_types=[
            pltpu.VMEM((qpw * k,), jnp.int32),
            pltpu.VMEM((chunk * k, _CPAD), jnp.float32),
            pltpu.VMEM((chunk, _CPAD), jnp.float32),
            pltpu.VMEM((chunk, _CPAD), jnp.float32),
            pltpu.VMEM((chunk, _CPAD), jnp.float32),
            pltpu.VMEM((chunk, _CPAD), jnp.float32),
            pltpu.SemaphoreType.DMA,
        ],
    )
    def sck(g_hbm, idx_hbm, mx_hbm, mn_hbm, sm_hbm, sq_hbm,
            idx_v, rows_v, omx, omn, osm, osq, sem):
        wid = jax.lax.axis_index("s") * nc + jax.lax.axis_index("c")
        baseq = wid * qpw
        pltpu.sync_copy(idx_hbm.at[pl.ds(baseq * k, qpw * k)], idx_v)

        def chunk_body(ch, carry):
            pltpu.async_copy(
                g_hbm.at[idx_v.at[pl.ds(ch * (chunk * k), chunk * k)]],
                rows_v, sem).wait()

            def q_body(q, c2):
                rbase = q * k
                for c in range(_CPAD // 16):
                    sl = pl.ds(c * 16, 16)
                    v = rows_v[rbase, sl]
                    mx = v
                    mn = v
                    sm = v
                    sq = v * v
                    for r in range(1, k):
                        v = rows_v[rbase + r, sl]
                        mx = jnp.maximum(mx, v)
                        mn = jnp.minimum(mn, v)
                        sm = sm + v
                        sq = sq + v * v
                    omx[q, sl] = mx
                    omn[q, sl] = mn
                    osm[q, sl] = sm
                    osq[q, sl] = sq
                return c2

            jax.lax.fori_loop(0, chunk, q_body, 0)
            row0 = baseq + ch * chunk
            pltpu.sync_copy(omx, mx_hbm.at[pl.ds(row0, chunk)])
            pltpu.sync_copy(omn, mn_hbm.at[pl.ds(row0, chunk)])
            pltpu.sync_copy(osm, sm_hbm.at[pl.ds(row0, chunk)])
            pltpu.sync_copy(osq, sq_hbm.at[pl.ds(row0, chunk)])
            return carry

        jax.lax.fori_loop(0, nchunks, chunk_body, 0)

    return sck


def _stats_body(sm_ref, sq_ref, nx_ref, ny_ref, nz_ref, wt_ref, st_ref, *, k):
    b = pl.program_id(0)
    qw = (nx_ref[0] * wt_ref[0:1, :] + ny_ref[0] * wt_ref[1:2, :]
          + nz_ref[0] * wt_ref[2:3, :])
    sm = sm_ref[0]
    sq = sq_ref[0]
    lin_sum = sm - k * qw
    lin_sq = sq - 2.0 * qw * sm + k * (qw * qw)

    @pl.when(b == 0)
    def _():
        st_ref[...] = jnp.zeros((8, _CPAD), jnp.float32)

    st_ref[0:1, :] += jnp.sum(lin_sum, axis=0, keepdims=True)
    st_ref[1:2, :] += jnp.sum(lin_sq, axis=0, keepdims=True)


def _stats(smg, sqg, nx3, ny3, nz3, wt_pad, k):
    b, s, _ = smg.shape
    m_spec = pl.BlockSpec((1, s, _CPAD), lambda i: (i, 0, 0))
    q_spec = pl.BlockSpec((1, s, 1), lambda i: (i, 0, 0))
    w_spec = pl.BlockSpec((_CPAD, _CPAD), lambda i: (0, 0))
    s_spec = pl.BlockSpec((8, _CPAD), lambda i: (0, 0))
    return pl.pallas_call(
        functools.partial(_stats_body, k=float(k)),
        grid=(b,),
        in_specs=[m_spec, m_spec, q_spec, q_spec, q_spec, w_spec],
        out_specs=s_spec,
        out_shape=jax.ShapeDtypeStruct((8, _CPAD), jnp.float32),
    )(smg, sqg, nx3, ny3, nz3, wt_pad)


def _apply_body(mx_ref, mn_ref, nx_ref, ny_ref, nz_ref, wt_ref, st_ref,
                gb_ref, out_ref, *, cnt):
    qw = (nx_ref[0] * wt_ref[0:1, :] + ny_ref[0] * wt_ref[1:2, :]
          + nz_ref[0] * wt_ref[2:3, :])
    mean = st_ref[0:1, :] / cnt
    ex2 = st_ref[1:2, :] / cnt
    var = ex2 - mean * mean
    gamma = gb_ref[0:1, :]
    beta = gb_ref[1:2, :]
    inv = jax.lax.rsqrt(var + _EPS)
    scale = gamma * inv
    shift = beta - mean * scale
    ext = jnp.where(gamma >= 0.0, mx_ref[0], mn_ref[0]) - qw
    out_ref[0] = jnp.maximum(ext * scale + shift, 0.0)


def _apply(mxg, mng, nx3, ny3, nz3, wt_pad, stats, gb, cnt):
    b, s, _ = mxg.shape
    m_spec = pl.BlockSpec((1, s, _CPAD), lambda i: (i, 0, 0))
    q_spec = pl.BlockSpec((1, s, 1), lambda i: (i, 0, 0))
    w_spec = pl.BlockSpec((_CPAD, _CPAD), lambda i: (0, 0))
    s_spec = pl.BlockSpec((8, _CPAD), lambda i: (0, 0))
    return pl.pallas_call(
        functools.partial(_apply_body, cnt=cnt),
        grid=(b,),
        in_specs=[m_spec, m_spec, q_spec, q_spec, q_spec, w_spec,
                  s_spec, s_spec],
        out_specs=m_spec,
        out_shape=jax.ShapeDtypeStruct((b, s, _CPAD), jnp.float32),
    )(mxg, mng, nx3, ny3, nz3, wt_pad, stats, gb)


def kernel(xyz, features, temb, W, gamma, beta):
    b, n, _ = xyz.shape
    cin = features.shape[-1]
    cout = W.shape[0]
    s = n // _STRIDE
    rows = n // 128

    xs = xyz[:, :, 0].reshape(b, rows, 128)
    ys = xyz[:, :, 1].reshape(b, rows, 128)
    zs = xyz[:, :, 2].reshape(b, rows, 128)
    nx, ny, nz = _fps(xs, ys, zs, n, s)
    new_xyz = jnp.stack(
        [nx.reshape(b, s), ny.reshape(b, s), nz.reshape(b, s)], axis=-1)

    pad = jnp.zeros((b, n, _CPAD - 3 - cin), jnp.float32)
    xf = jnp.concatenate([xyz, features, pad], axis=-1)
    wt_pad = jnp.concatenate(
        [W.T, jnp.zeros((_CPAD - 3 - cin, cout), jnp.float32)], axis=0)
    wt_pad = jnp.concatenate(
        [wt_pad, jnp.zeros((_CPAD, _CPAD - cout), jnp.float32)], axis=1)
    g = _lin(xf, wt_pad)

    nx3 = nx.reshape(b, s, 1)
    ny3 = ny.reshape(b, s, 1)
    nz3 = nz.reshape(b, s, 1)
    xs3 = xyz[:, :, 0].reshape(b, 1, n)
    ys3 = xyz[:, :, 1].reshape(b, 1, n)
    zs3 = xyz[:, :, 2].reshape(b, 1, n)
    knn_idx = _knn(nx3, ny3, nz3, xs3, ys3, zs3, n, s, _K)

    sck = _sc_gather_build(b * s, _K)
    mxg, mng, smg, sqg = sck(g.reshape(b * n, _CPAD),
                             knn_idx.reshape(b * s * _K))
    mxg = mxg.reshape(b, s, _CPAD)
    mng = mng.reshape(b, s, _CPAD)
    smg = smg.reshape(b, s, _CPAD)
    sqg = sqg.reshape(b, s, _CPAD)

    stats = _stats(smg, sqg, nx3, ny3, nz3, wt_pad, _K)
    gb = jnp.concatenate(
        [gamma.reshape(1, -1), beta.reshape(1, -1),
         jnp.zeros((6, cout), jnp.float32)], axis=0)
    gb = jnp.concatenate(
        [gb, jnp.zeros((8, _CPAD - cout), jnp.float32)], axis=1)
    cnt = float(b * s * _K)
    feats_out = _apply(mxg, mng, nx3, ny3, nz3, wt_pad, stats, gb,
                       cnt)[:, :, :cout]

    return new_xyz, feats_out, temb[:, :s, :]
